# baseline (speedup 1.0000x reference)
import functools

import jax
import jax.numpy as jnp
from jax import lax
from jax.experimental import pallas as pl
from jax.experimental.pallas import tpu as pltpu
from jax.experimental.pallas import tpu_sc as plsc

HIDE_DIM = 32
LANES = 16
NUM_CORES = 2
NUM_SUBCORES = 16
NW = NUM_CORES * NUM_SUBCORES
CHUNK = 128


@functools.lru_cache(maxsize=None)
def _build(batch: int):
    b_per_w = batch // NW
    n_chunks = b_per_w // CHUNK
    mesh = plsc.VectorSubcoreMesh(core_axis_name="c", subcore_axis_name="s")

    @functools.partial(
        pl.kernel,
        mesh=mesh,
        compiler_params=pltpu.CompilerParams(use_tc_tiling_on_sc=False),
        out_type=jax.ShapeDtypeStruct((batch,), jnp.float32),
        scratch_types=[
            pltpu.VMEM((n_chunks, CHUNK), jnp.int32),
            pltpu.VMEM((n_chunks, CHUNK), jnp.int32),
            pltpu.VMEM((b_per_w, HIDE_DIM), jnp.float32),
            pltpu.VMEM((b_per_w, HIDE_DIM), jnp.float32),
            pltpu.VMEM((b_per_w,), jnp.float32),
            pltpu.SemaphoreType.DMA,
            pltpu.SemaphoreType.DMA,
        ],
    )
    def k(uidx_hbm, iidx_hbm, utab_hbm, itab_hbm, out_hbm,
          uidx_v, iidx_v, urows_v, irows_v, out_v, sem_u, sem_i):
        wid = lax.axis_index("s") * NUM_CORES + lax.axis_index("c")
        base = wid * b_per_w
        pltpu.sync_copy(uidx_hbm.at[pl.ds(wid * n_chunks, n_chunks)], uidx_v)
        pltpu.sync_copy(iidx_hbm.at[pl.ds(wid * n_chunks, n_chunks)], iidx_v)

        copies = []
        for j in range(n_chunks):
            copies.append(pltpu.async_copy(
                utab_hbm.at[uidx_v.at[j]],
                urows_v.at[pl.ds(j * CHUNK, CHUNK)], sem_u))
            copies.append(pltpu.async_copy(
                itab_hbm.at[iidx_v.at[j]],
                irows_v.at[pl.ds(j * CHUNK, CHUNK)], sem_i))
        for c in copies:
            c.wait()

        lane = lax.iota(jnp.int32, LANES)
        perms = [lane ^ k for k in (8, 4, 2, 1)]

        def rowsum(r):
            u0 = urows_v[r, pl.ds(0, LANES)]
            u1 = urows_v[r, pl.ds(LANES, LANES)]
            i0 = irows_v[r, pl.ds(0, LANES)]
            i1 = irows_v[r, pl.ds(LANES, LANES)]
            v = u0 * i0 + u1 * i1
            for p in perms:
                v = v + v.at[p].get(mode="promise_in_bounds")
            return v

        def body(g, _):
            acc = jnp.zeros((LANES,), jnp.float32)
            r0 = g * LANES
            for r in range(LANES):
                acc = jnp.where(lane == r, rowsum(r0 + r), acc)
            out_v[pl.ds(r0, LANES)] = acc
            return 0

        lax.fori_loop(0, b_per_w // LANES, body, 0)
        pltpu.sync_copy(out_v, out_hbm.at[pl.ds(base, b_per_w)])

    return k


def kernel(user_idx, item_idx, userEmbed, itemEmbed):
    batch = user_idx.shape[0]
    b_per_w = batch // NW
    n_chunks = b_per_w // CHUNK
    uidx2 = user_idx.reshape(NW * n_chunks, CHUNK)
    iidx2 = item_idx.reshape(NW * n_chunks, CHUNK)
    return _build(batch)(uidx2, iidx2, userEmbed, itemEmbed)
